# baseline (speedup 1.0000x reference)
import numpy as np
from functools import partial
import jax
import jax.numpy as jnp
from jax import lax
from jax.experimental import pallas as pl
from jax.experimental.pallas import tpu as pltpu
from jax.experimental.pallas import tpu_sc as plsc

N = 50000
E = 800000
H = 128
L = 4
EDGE_IN = 2
MASK_RATIO = 0.15
NUM_MASK = max(1, int(E * MASK_RATIO))

BE = 6400
BN = 2000


def _mask_constants():
    cpu = jax.devices("cpu")[0]
    with jax.default_device(cpu):
        mkey = jax.random.key(42)
        k1, k2, k3 = jax.random.split(mkey, 3)
        perm = jax.random.permutation(k1, E)
        mask_indices = np.asarray(perm[:NUM_MASK])
        rand = np.asarray(jax.random.uniform(k2, (NUM_MASK,)))
        rand_feat = np.asarray(
            jax.random.normal(k3, (NUM_MASK, EDGE_IN), dtype=jnp.float32))
    is_tok = np.zeros((E, 1), np.float32)
    rand_const = np.zeros((E, EDGE_IN), np.float32)
    is_tok[mask_indices[rand < 0.8], 0] = 1.0
    sel = (rand >= 0.8) & (rand < 0.9)
    rand_const[mask_indices[sel]] = rand_feat[sel]
    use_rand = np.zeros((E, 1), np.float32)
    use_rand[mask_indices[sel], 0] = 1.0
    w_loss = np.zeros((E, 1), np.float32)
    w_loss[mask_indices, 0] = 1.0 / (NUM_MASK * EDGE_IN)
    return is_tok, use_rand, rand_const, w_loss


_IS_TOK, _USE_RAND, _RAND_CONST, _W_LOSS = _mask_constants()

_SW = E // 32
_SC_C = 200
_SNCH = _SW // _SC_C
_NPAD = 8


def _sc_segsum_body(m_hbm, dst_hbm, agg_hbm, brow_hbm, bid_hbm,
                    mva, mvb, rb0, rb1, bb0, bb1, bidv, dsm, fsm, sem, sf):
    wid = lax.axis_index("s") * _NC + lax.axis_index("c")
    base = wid * _SW
    lanes = lax.iota(jnp.int32, 16)
    lanesf = [lanes + k * 16 for k in range(8)]
    zv = jnp.zeros((16,), jnp.float32)
    for k in range(8):
        bb0[pl.ds(k * 16, 16)] = zv

    pltpu.sync_copy(dst_hbm.at[pl.ds(base, 8)], fsm.at[pl.ds(0, 8)])
    first_own = fsm[pl.ds(0, 16)][0]
    padid = N + jnp.bitwise_and(wid, _NPAD - 1)

    def _mdesc(buf):
        mv = mva if buf == 0 else mvb
        return pltpu.make_async_copy(
            m_hbm.at[pl.ds(0, _SC_C * H)], mv, sem.at[buf])

    def _fdesc(buf):
        rb = rb0 if buf == 0 else rb1
        return pltpu.make_async_copy(rb, agg_hbm.at[pl.ds(0, H)],
                                     sf.at[buf])

    def _start0(j):
        o = base + j * _SC_C
        pltpu.sync_copy(dst_hbm.at[pl.ds(o, _SC_C)],
                        dsm.at[pl.ds(0, _SC_C)])
        pltpu.make_async_copy(m_hbm.at[pl.ds(o * H, _SC_C * H)], mva,
                              sem.at[0]).start()

    def _start1(j):
        o = base + j * _SC_C
        pltpu.sync_copy(dst_hbm.at[pl.ds(o, _SC_C)],
                        dsm.at[pl.ds(_SC_C, _SC_C)])
        pltpu.make_async_copy(m_hbm.at[pl.ds(o * H, _SC_C * H)], mvb,
                              sem.at[1]).start()

    _start0(0)

    def edge_loop(mv, doff, c):
        def edge_body(mv, doff, e, ecarry):
            prev, flushno, ifl, acc = ecarry
            d = dsm[pl.ds(doff + e, 16)][0]
            row = tuple(mv[pl.ds(e * H + k * 16, 16)] for k in range(8))
            is_new = d != prev
            intr = is_new & (flushno > 0)
            par = jnp.bitwise_and(ifl, 1)

            @pl.when(intr & (par == 0))
            def _():
                @pl.when(ifl >= 2)
                def _():
                    _fdesc(0).wait()

                for k in range(8):
                    rb0[pl.ds(k * 16, 16)] = acc[k]
                pltpu.make_async_copy(
                    rb0, agg_hbm.at[pl.ds(prev * H, H)], sf.at[0]).start()

            @pl.when(intr & (par == 1))
            def _():
                @pl.when(ifl >= 2)
                def _():
                    _fdesc(1).wait()

                for k in range(8):
                    rb1[pl.ds(k * 16, 16)] = acc[k]
                pltpu.make_async_copy(
                    rb1, agg_hbm.at[pl.ds(prev * H, H)], sf.at[1]).start()

            @pl.when(is_new & (flushno == 0))
            def _():
                for k in range(8):
                    bb0[pl.ds(k * 16, 16)] = acc[k]

            flushno2 = jnp.where(is_new, flushno + 1, flushno)
            ifl2 = jnp.where(intr, ifl + 1, ifl)
            acc2 = tuple(jnp.where(is_new, row[k], acc[k] + row[k])
                         for k in range(8))
            return (d, flushno2, ifl2, acc2)

        return lax.fori_loop(0, _SC_C, partial(edge_body, mv, doff), c)

    def pair_body(jj, carry):
        a = 2 * jj
        _start1(a + 1)
        _mdesc(0).wait()
        carry = edge_loop(mva, 0, carry)

        @pl.when(a + 2 < _SNCH)
        def _():
            _start0(a + 2)

        _mdesc(1).wait()
        carry = edge_loop(mvb, _SC_C, carry)
        return carry

    acc0 = tuple(jnp.zeros((16,), jnp.float32) for _ in range(8))
    init = (first_own, jnp.int32(0), jnp.int32(0), acc0)
    carry = lax.fori_loop(0, _SNCH // 2, pair_body, init)
    _mdesc(0).wait()
    prev, flushno, ifl, acc = edge_loop(mva, 0, carry)

    @pl.when(ifl >= 1)
    def _():
        @pl.when(jnp.bitwise_and(ifl - 1, 1) == 0)
        def _():
            _fdesc(0).wait()

        @pl.when(jnp.bitwise_and(ifl - 1, 1) == 1)
        def _():
            _fdesc(1).wait()

    @pl.when(ifl >= 2)
    def _():
        @pl.when(jnp.bitwise_and(ifl, 1) == 0)
        def _():
            _fdesc(0).wait()

        @pl.when(jnp.bitwise_and(ifl, 1) == 1)
        def _():
            _fdesc(1).wait()

    for k in range(8):
        bb1[pl.ds(k * 16, 16)] = acc[k]
    pltpu.sync_copy(bb0, brow_hbm.at[pl.ds(wid * 2 * H, H)])
    pltpu.sync_copy(bb1, brow_hbm.at[pl.ds(wid * 2 * H + H, H)])

    first_id = jnp.where(flushno > 0, first_own, padid)
    idrow = jnp.where(lanes == 0, first_id,
                      jnp.where(lanes == 1, prev, padid))
    bidv[pl.ds(0, 16)] = idrow
    pltpu.sync_copy(bidv, bid_hbm.at[pl.ds(wid * 16, 16)])


def _sc_segsum(m, sdst, agg_ref):
    mesh = plsc.VectorSubcoreMesh(core_axis_name="c", subcore_axis_name="s")
    f = pl.kernel(
        _sc_segsum_body,
        mesh=mesh,
        out_type=(
            jax.ShapeDtypeStruct((64 * H,), jnp.float32),
            jax.ShapeDtypeStruct((32 * 16,), jnp.int32),
        ),
        scratch_types=[
            pltpu.VMEM((_SC_C * H,), jnp.float32),
            pltpu.VMEM((_SC_C * H,), jnp.float32),
            pltpu.VMEM((H,), jnp.float32),
            pltpu.VMEM((H,), jnp.float32),
            pltpu.VMEM((H,), jnp.float32),
            pltpu.VMEM((H,), jnp.float32),
            pltpu.VMEM((16,), jnp.int32),
            pltpu.VMEM((2 * _SC_C + 16,), jnp.int32),
            pltpu.VMEM((16,), jnp.int32),
            pltpu.SemaphoreType.DMA((2,)),
            pltpu.SemaphoreType.DMA((2,)),
        ],
    )
    return f(m, sdst, agg_ref)


_NC = 2
_NS = 16
_NW = _NC * _NS
_GC = 400


def _sc_gather_body(nrows, table_hbm, idx_hbm, out_hbm, idx_a, idx_b,
                    rows_v, sem):
    wid = lax.axis_index("s") * _NC + lax.axis_index("c")
    per_w = nrows // _NW
    nchunk = per_w // _GC
    base = wid * per_w

    def _desc(buf):
        idx = idx_a if buf == 0 else idx_b
        return pltpu.make_async_copy(table_hbm.at[idx], rows_v.at[buf],
                                     sem.at[buf])

    def _start(j):
        @pl.when(j % 2 == 0)
        def _():
            pltpu.sync_copy(idx_hbm.at[pl.ds(base + j * _GC, _GC)], idx_a)
            _desc(0).start()

        @pl.when(j % 2 == 1)
        def _():
            pltpu.sync_copy(idx_hbm.at[pl.ds(base + j * _GC, _GC)], idx_b)
            _desc(1).start()

    def _wait(j):
        @pl.when(j % 2 == 0)
        def _():
            _desc(0).wait()

        @pl.when(j % 2 == 1)
        def _():
            _desc(1).wait()

    _start(0)

    def body(j, carry):
        nxt = j + 1

        @pl.when(nxt < nchunk)
        def _():
            _start(nxt)

        _wait(j)
        pltpu.sync_copy(rows_v.at[j % 2],
                        out_hbm.at[pl.ds(base + j * _GC, _GC)])
        return carry

    lax.fori_loop(0, nchunk, body, 0)


def _sc_gather(table, idx, nrows):
    mesh = plsc.VectorSubcoreMesh(core_axis_name="c", subcore_axis_name="s")
    f = partial(
        pl.kernel,
        mesh=mesh,
        out_type=jax.ShapeDtypeStruct((nrows, H), jnp.float32),
        scratch_types=[
            pltpu.VMEM((_GC,), jnp.int32),
            pltpu.VMEM((_GC,), jnp.int32),
            pltpu.VMEM((2, _GC, H), jnp.float32),
            pltpu.SemaphoreType.DMA((2,)),
        ],
        compiler_params=pltpu.CompilerParams(use_tc_tiling_on_sc=True),
    )(partial(_sc_gather_body, nrows))
    return f(table, idx)


def _msg_body(hs_ref, hd_ref, attr_ref, ws_ref, wd_ref, we_ref, bm_ref,
              wedge_ref, bedge_ref, m_ref):
    e = jnp.maximum(attr_ref[...] @ wedge_ref[...] + bedge_ref[...], 0.0)
    acc = hs_ref[...] @ ws_ref[...]
    acc = acc + hd_ref[...] @ wd_ref[...]
    acc = acc + e @ we_ref[...]
    m_ref[...] = jnp.maximum(acc + bm_ref[...], 0.0).reshape(BE * H)


def _msg_matmul(gath, attr, Ws, Wd, We, bm, Wedge, bedge):
    grid = (E // BE,)
    nblk = E // BE
    ein = lambda i: (i, 0)
    hd_map = lambda i: (i + nblk, 0)
    full = lambda i: (0, 0)
    return pl.pallas_call(
        _msg_body,
        grid=grid,
        in_specs=[
            pl.BlockSpec((BE, H), ein),
            pl.BlockSpec((BE, H), hd_map),
            pl.BlockSpec((BE, EDGE_IN), ein),
            pl.BlockSpec((H, H), full),
            pl.BlockSpec((H, H), full),
            pl.BlockSpec((H, H), full),
            pl.BlockSpec((1, H), full),
            pl.BlockSpec((EDGE_IN, H), full),
            pl.BlockSpec((1, H), full),
        ],
        out_specs=pl.BlockSpec((BE * H,), lambda i: (i,)),
        out_shape=jax.ShapeDtypeStruct((E * H,), jnp.float32),
        compiler_params=pltpu.CompilerParams(
            dimension_semantics=("arbitrary",)),
    )(gath, gath, attr, Ws, Wd, We, bm, Wedge, bedge)


def _upd_body(h_ref, agg_ref, wu1_ref, wu2_ref, bu_ref, out_ref):
    acc = h_ref[...] @ wu1_ref[...]
    acc = acc + agg_ref[...] @ wu2_ref[...]
    out_ref[...] = h_ref[...] + jnp.maximum(acc + bu_ref[...], 0.0)


def _update(h, agg, Wu1, Wu2, bu):
    grid = (N // BN,)
    ein = lambda i: (i, 0)
    full = lambda i: (0, 0)
    return pl.pallas_call(
        _upd_body,
        grid=grid,
        in_specs=[
            pl.BlockSpec((BN, H), ein),
            pl.BlockSpec((BN, H), ein),
            pl.BlockSpec((H, H), full),
            pl.BlockSpec((H, H), full),
            pl.BlockSpec((1, H), full),
        ],
        out_specs=pl.BlockSpec((BN, H), ein),
        out_shape=jax.ShapeDtypeStruct((N, H), jnp.float32),
        compiler_params=pltpu.CompilerParams(
            dimension_semantics=("arbitrary",)),
    )(h, agg, Wu1, Wu2, bu)


def _head_body(a_ref, b_ref, attr_ref, w_ref, b1_ref, w2_ref, b2_ref,
               out_ref, loss_ref):
    i = pl.program_id(0)

    @pl.when(i == 0)
    def _init():
        loss_ref[...] = jnp.zeros_like(loss_ref)

    hid = jnp.maximum(a_ref[...] + b_ref[...] + b1_ref[...], 0.0)
    rec = hid @ w2_ref[...] + b2_ref[...]
    out_ref[...] = rec
    d = rec - attr_ref[...]
    part = jnp.sum(d * d * w_ref[...], axis=0, keepdims=True)
    loss_ref[...] += part


def _head(gath, attr, b1, W2, b2):
    grid = (E // BE,)
    nblk = E // BE
    ein = lambda i: (i, 0)
    hd_map = lambda i: (i + nblk, 0)
    full = lambda i: (0, 0)
    return pl.pallas_call(
        _head_body,
        grid=grid,
        in_specs=[
            pl.BlockSpec((BE, H), ein),
            pl.BlockSpec((BE, H), hd_map),
            pl.BlockSpec((BE, EDGE_IN), ein),
            pl.BlockSpec((BE, 1), ein),
            pl.BlockSpec((1, H), full),
            pl.BlockSpec((H, EDGE_IN), full),
            pl.BlockSpec((1, EDGE_IN), full),
        ],
        out_specs=[
            pl.BlockSpec((BE, EDGE_IN), ein),
            pl.BlockSpec((1, EDGE_IN), full),
        ],
        out_shape=[
            jax.ShapeDtypeStruct((E, EDGE_IN), jnp.float32),
            jax.ShapeDtypeStruct((1, EDGE_IN), jnp.float32),
        ],
        compiler_params=pltpu.CompilerParams(
            dimension_semantics=("arbitrary",)),
    )(gath, gath, attr, jnp.asarray(_W_LOSS), b1, W2, b2)


def _proj_body(h_ref, w1_ref, out_ref):
    out_ref[...] = h_ref[...] @ w1_ref[0]


def _proj(node_emb, W1ab):
    nblk = N // BN
    grid = (2, nblk)
    return pl.pallas_call(
        _proj_body,
        grid=grid,
        in_specs=[
            pl.BlockSpec((BN, H), lambda g, i: (i, 0)),
            pl.BlockSpec((1, H, H), lambda g, i: (g, 0, 0)),
        ],
        out_specs=pl.BlockSpec((BN, H), lambda g, i: (g * nblk + i, 0)),
        out_shape=jax.ShapeDtypeStruct((2 * N, H), jnp.float32),
        compiler_params=pltpu.CompilerParams(
            dimension_semantics=("arbitrary", "arbitrary")),
    )(node_emb, W1ab)


def kernel(x, edge_index, edge_attr, W_node, b_node, W_edge, b_edge,
           W_msg, b_msg, W_upd, b_upd, W_h1, b_h1, W_h2, b_h2, mask_token):
    src = edge_index[0]
    dst = edge_index[1]

    tok = jnp.asarray(_IS_TOK)
    urand = jnp.asarray(_USE_RAND)
    masked_attr = (1.0 - tok - urand) * edge_attr \
        + tok * mask_token[None, :] + urand * jnp.asarray(_RAND_CONST)

    iota = lax.iota(jnp.int32, E)
    sdst, perm = lax.sort_key_val(dst, iota)
    ssrc = jnp.take(src, perm, axis=0, mode='clip')
    attr_s = jnp.take(masked_attr, perm, axis=0, mode='clip')

    idx_cat = jnp.concatenate([ssrc, sdst])
    idx_head = jnp.concatenate([src, dst + N])

    h = jax.nn.relu(x @ W_node + b_node)

    b_msg2 = b_msg.reshape(L, 1, H)
    b_upd2 = b_upd.reshape(L, 1, H)
    bedge2 = b_edge.reshape(1, H)

    for l in range(L):
        gath = _sc_gather(h, idx_cat, 2 * E)
        Ws = W_msg[l, :H, :]
        Wd = W_msg[l, H:2 * H, :]
        We = W_msg[l, 2 * H:, :]
        m = _msg_matmul(gath, attr_s, Ws, Wd, We,
                        b_msg2[l], W_edge, bedge2)
        agg_ref = jax.new_ref(jnp.zeros(((N + _NPAD) * H,), jnp.float32))
        brow, bid = _sc_segsum(m, sdst, agg_ref)
        bid2 = bid.reshape(32, 16)
        br3 = brow.reshape(32, 2, H)
        agg_p = agg_ref[...].reshape(N + _NPAD, H) \
            .at[bid2[:, 0]].add(br3[:, 0, :]) \
            .at[bid2[:, 1]].add(br3[:, 1, :])
        h = _update(h, agg_p, W_upd[l, :H, :], W_upd[l, H:, :], b_upd2[l])

    node_emb = h

    projcat = _proj(node_emb, jnp.stack([W_h1[:H, :], W_h1[H:, :]]))
    gath_head = _sc_gather(projcat, idx_head, 2 * E)
    reconstructed, loss_part = _head(gath_head, edge_attr,
                                     b_h1.reshape(1, H), W_h2,
                                     b_h2.reshape(1, EDGE_IN))
    loss = jnp.sum(loss_part)
    return (loss, reconstructed, node_emb)


# speedup vs baseline: 1.0030x; 1.0030x over previous
import numpy as np
from functools import partial
import jax
import jax.numpy as jnp
from jax import lax
from jax.experimental import pallas as pl
from jax.experimental.pallas import tpu as pltpu
from jax.experimental.pallas import tpu_sc as plsc

N = 50000
E = 800000
H = 128
L = 4
EDGE_IN = 2
MASK_RATIO = 0.15
NUM_MASK = max(1, int(E * MASK_RATIO))

BE = 8000
BN = 2000


def _mask_constants():
    cpu = jax.devices("cpu")[0]
    with jax.default_device(cpu):
        mkey = jax.random.key(42)
        k1, k2, k3 = jax.random.split(mkey, 3)
        perm = jax.random.permutation(k1, E)
        mask_indices = np.asarray(perm[:NUM_MASK])
        rand = np.asarray(jax.random.uniform(k2, (NUM_MASK,)))
        rand_feat = np.asarray(
            jax.random.normal(k3, (NUM_MASK, EDGE_IN), dtype=jnp.float32))
    is_tok = np.zeros((E, 1), np.float32)
    rand_const = np.zeros((E, EDGE_IN), np.float32)
    is_tok[mask_indices[rand < 0.8], 0] = 1.0
    sel = (rand >= 0.8) & (rand < 0.9)
    rand_const[mask_indices[sel]] = rand_feat[sel]
    use_rand = np.zeros((E, 1), np.float32)
    use_rand[mask_indices[sel], 0] = 1.0
    w_loss = np.zeros((E, 1), np.float32)
    w_loss[mask_indices, 0] = 1.0 / (NUM_MASK * EDGE_IN)
    return is_tok, use_rand, rand_const, w_loss


_IS_TOK, _USE_RAND, _RAND_CONST, _W_LOSS = _mask_constants()

_SW = E // 32
_SC_C = 200
_SNCH = _SW // _SC_C
_NPAD = 8


def _sc_segsum_body(m_hbm, dst_hbm, agg_hbm, brow_hbm, bid_hbm,
                    mva, mvb, rb0, rb1, bb0, bb1, bidv, dsm, fsm, sem, sf):
    wid = lax.axis_index("s") * _NC + lax.axis_index("c")
    base = wid * _SW
    lanes = lax.iota(jnp.int32, 16)
    lanesf = [lanes + k * 16 for k in range(8)]
    zv = jnp.zeros((16,), jnp.float32)
    for k in range(8):
        bb0[pl.ds(k * 16, 16)] = zv

    pltpu.sync_copy(dst_hbm.at[pl.ds(base, 8)], fsm.at[pl.ds(0, 8)])
    first_own = fsm[pl.ds(0, 16)][0]
    padid = N + jnp.bitwise_and(wid, _NPAD - 1)

    def _mdesc(buf):
        mv = mva if buf == 0 else mvb
        return pltpu.make_async_copy(
            m_hbm.at[pl.ds(0, _SC_C * H)], mv, sem.at[buf])

    def _fdesc(buf):
        rb = rb0 if buf == 0 else rb1
        return pltpu.make_async_copy(rb, agg_hbm.at[pl.ds(0, H)],
                                     sf.at[buf])

    def _start0(j):
        o = base + j * _SC_C
        pltpu.sync_copy(dst_hbm.at[pl.ds(o, _SC_C)],
                        dsm.at[pl.ds(0, _SC_C)])
        pltpu.make_async_copy(m_hbm.at[pl.ds(o * H, _SC_C * H)], mva,
                              sem.at[0]).start()

    def _start1(j):
        o = base + j * _SC_C
        pltpu.sync_copy(dst_hbm.at[pl.ds(o, _SC_C)],
                        dsm.at[pl.ds(_SC_C, _SC_C)])
        pltpu.make_async_copy(m_hbm.at[pl.ds(o * H, _SC_C * H)], mvb,
                              sem.at[1]).start()

    _start0(0)

    def edge_loop(mv, doff, c):
        def edge_body(mv, doff, e, ecarry):
            prev, flushno, ifl, acc = ecarry
            d = dsm[pl.ds(doff + e, 16)][0]
            row = tuple(mv[pl.ds(e * H + k * 16, 16)] for k in range(8))
            is_new = d != prev
            intr = is_new & (flushno > 0)
            par = jnp.bitwise_and(ifl, 1)

            @pl.when(intr & (par == 0))
            def _():
                @pl.when(ifl >= 2)
                def _():
                    _fdesc(0).wait()

                for k in range(8):
                    rb0[pl.ds(k * 16, 16)] = acc[k]
                pltpu.make_async_copy(
                    rb0, agg_hbm.at[pl.ds(prev * H, H)], sf.at[0]).start()

            @pl.when(intr & (par == 1))
            def _():
                @pl.when(ifl >= 2)
                def _():
                    _fdesc(1).wait()

                for k in range(8):
                    rb1[pl.ds(k * 16, 16)] = acc[k]
                pltpu.make_async_copy(
                    rb1, agg_hbm.at[pl.ds(prev * H, H)], sf.at[1]).start()

            @pl.when(is_new & (flushno == 0))
            def _():
                for k in range(8):
                    bb0[pl.ds(k * 16, 16)] = acc[k]

            flushno2 = jnp.where(is_new, flushno + 1, flushno)
            ifl2 = jnp.where(intr, ifl + 1, ifl)
            acc2 = tuple(jnp.where(is_new, row[k], acc[k] + row[k])
                         for k in range(8))
            return (d, flushno2, ifl2, acc2)

        return lax.fori_loop(0, _SC_C, partial(edge_body, mv, doff), c)

    def pair_body(jj, carry):
        a = 2 * jj
        _start1(a + 1)
        _mdesc(0).wait()
        carry = edge_loop(mva, 0, carry)

        @pl.when(a + 2 < _SNCH)
        def _():
            _start0(a + 2)

        _mdesc(1).wait()
        carry = edge_loop(mvb, _SC_C, carry)
        return carry

    acc0 = tuple(jnp.zeros((16,), jnp.float32) for _ in range(8))
    init = (first_own, jnp.int32(0), jnp.int32(0), acc0)
    carry = lax.fori_loop(0, _SNCH // 2, pair_body, init)
    _mdesc(0).wait()
    prev, flushno, ifl, acc = edge_loop(mva, 0, carry)

    @pl.when(ifl >= 1)
    def _():
        @pl.when(jnp.bitwise_and(ifl - 1, 1) == 0)
        def _():
            _fdesc(0).wait()

        @pl.when(jnp.bitwise_and(ifl - 1, 1) == 1)
        def _():
            _fdesc(1).wait()

    @pl.when(ifl >= 2)
    def _():
        @pl.when(jnp.bitwise_and(ifl, 1) == 0)
        def _():
            _fdesc(0).wait()

        @pl.when(jnp.bitwise_and(ifl, 1) == 1)
        def _():
            _fdesc(1).wait()

    for k in range(8):
        bb1[pl.ds(k * 16, 16)] = acc[k]
    pltpu.sync_copy(bb0, brow_hbm.at[pl.ds(wid * 2 * H, H)])
    pltpu.sync_copy(bb1, brow_hbm.at[pl.ds(wid * 2 * H + H, H)])

    first_id = jnp.where(flushno > 0, first_own, padid)
    idrow = jnp.where(lanes == 0, first_id,
                      jnp.where(lanes == 1, prev, padid))
    bidv[pl.ds(0, 16)] = idrow
    pltpu.sync_copy(bidv, bid_hbm.at[pl.ds(wid * 16, 16)])


def _sc_segsum(m, sdst, agg_ref):
    mesh = plsc.VectorSubcoreMesh(core_axis_name="c", subcore_axis_name="s")
    f = pl.kernel(
        _sc_segsum_body,
        mesh=mesh,
        out_type=(
            jax.ShapeDtypeStruct((64 * H,), jnp.float32),
            jax.ShapeDtypeStruct((32 * 16,), jnp.int32),
        ),
        scratch_types=[
            pltpu.VMEM((_SC_C * H,), jnp.float32),
            pltpu.VMEM((_SC_C * H,), jnp.float32),
            pltpu.VMEM((H,), jnp.float32),
            pltpu.VMEM((H,), jnp.float32),
            pltpu.VMEM((H,), jnp.float32),
            pltpu.VMEM((H,), jnp.float32),
            pltpu.VMEM((16,), jnp.int32),
            pltpu.VMEM((2 * _SC_C + 16,), jnp.int32),
            pltpu.VMEM((16,), jnp.int32),
            pltpu.SemaphoreType.DMA((2,)),
            pltpu.SemaphoreType.DMA((2,)),
        ],
    )
    return f(m, sdst, agg_ref)


_NC = 2
_NS = 16
_NW = _NC * _NS
_GC = 400


def _sc_gather_body(nrows, table_hbm, idx_hbm, out_hbm, idx_a, idx_b,
                    rows_v, sem):
    wid = lax.axis_index("s") * _NC + lax.axis_index("c")
    per_w = nrows // _NW
    nchunk = per_w // _GC
    base = wid * per_w

    def _desc(buf):
        idx = idx_a if buf == 0 else idx_b
        return pltpu.make_async_copy(table_hbm.at[idx], rows_v.at[buf],
                                     sem.at[buf])

    def _start(j):
        @pl.when(j % 2 == 0)
        def _():
            pltpu.sync_copy(idx_hbm.at[pl.ds(base + j * _GC, _GC)], idx_a)
            _desc(0).start()

        @pl.when(j % 2 == 1)
        def _():
            pltpu.sync_copy(idx_hbm.at[pl.ds(base + j * _GC, _GC)], idx_b)
            _desc(1).start()

    def _wait(j):
        @pl.when(j % 2 == 0)
        def _():
            _desc(0).wait()

        @pl.when(j % 2 == 1)
        def _():
            _desc(1).wait()

    _start(0)

    def body(j, carry):
        nxt = j + 1

        @pl.when(nxt < nchunk)
        def _():
            _start(nxt)

        _wait(j)
        pltpu.sync_copy(rows_v.at[j % 2],
                        out_hbm.at[pl.ds(base + j * _GC, _GC)])
        return carry

    lax.fori_loop(0, nchunk, body, 0)


def _sc_gather(table, idx, nrows):
    mesh = plsc.VectorSubcoreMesh(core_axis_name="c", subcore_axis_name="s")
    f = partial(
        pl.kernel,
        mesh=mesh,
        out_type=jax.ShapeDtypeStruct((nrows, H), jnp.float32),
        scratch_types=[
            pltpu.VMEM((_GC,), jnp.int32),
            pltpu.VMEM((_GC,), jnp.int32),
            pltpu.VMEM((2, _GC, H), jnp.float32),
            pltpu.SemaphoreType.DMA((2,)),
        ],
        compiler_params=pltpu.CompilerParams(use_tc_tiling_on_sc=True),
    )(partial(_sc_gather_body, nrows))
    return f(table, idx)


def _msg_body(hs_ref, hd_ref, attr_ref, ws_ref, wd_ref, we_ref, bm_ref,
              wedge_ref, bedge_ref, m_ref):
    e = jnp.maximum(attr_ref[...] @ wedge_ref[...] + bedge_ref[...], 0.0)
    acc = hs_ref[...] @ ws_ref[...]
    acc = acc + hd_ref[...] @ wd_ref[...]
    acc = acc + e @ we_ref[...]
    m_ref[...] = jnp.maximum(acc + bm_ref[...], 0.0).reshape(BE * H)


def _msg_matmul(gath, attr, Ws, Wd, We, bm, Wedge, bedge):
    grid = (E // BE,)
    nblk = E // BE
    ein = lambda i: (i, 0)
    hd_map = lambda i: (i + nblk, 0)
    full = lambda i: (0, 0)
    return pl.pallas_call(
        _msg_body,
        grid=grid,
        in_specs=[
            pl.BlockSpec((BE, H), ein),
            pl.BlockSpec((BE, H), hd_map),
            pl.BlockSpec((BE, EDGE_IN), ein),
            pl.BlockSpec((H, H), full),
            pl.BlockSpec((H, H), full),
            pl.BlockSpec((H, H), full),
            pl.BlockSpec((1, H), full),
            pl.BlockSpec((EDGE_IN, H), full),
            pl.BlockSpec((1, H), full),
        ],
        out_specs=pl.BlockSpec((BE * H,), lambda i: (i,)),
        out_shape=jax.ShapeDtypeStruct((E * H,), jnp.float32),
        compiler_params=pltpu.CompilerParams(
            dimension_semantics=("arbitrary",)),
    )(gath, gath, attr, Ws, Wd, We, bm, Wedge, bedge)


def _upd_body(h_ref, agg_ref, wu1_ref, wu2_ref, bu_ref, out_ref):
    acc = h_ref[...] @ wu1_ref[...]
    acc = acc + agg_ref[...] @ wu2_ref[...]
    out_ref[...] = h_ref[...] + jnp.maximum(acc + bu_ref[...], 0.0)


def _update(h, agg, Wu1, Wu2, bu):
    grid = (N // BN,)
    ein = lambda i: (i, 0)
    full = lambda i: (0, 0)
    return pl.pallas_call(
        _upd_body,
        grid=grid,
        in_specs=[
            pl.BlockSpec((BN, H), ein),
            pl.BlockSpec((BN, H), ein),
            pl.BlockSpec((H, H), full),
            pl.BlockSpec((H, H), full),
            pl.BlockSpec((1, H), full),
        ],
        out_specs=pl.BlockSpec((BN, H), ein),
        out_shape=jax.ShapeDtypeStruct((N, H), jnp.float32),
        compiler_params=pltpu.CompilerParams(
            dimension_semantics=("arbitrary",)),
    )(h, agg, Wu1, Wu2, bu)


def _head_body(a_ref, b_ref, attr_ref, w_ref, b1_ref, w2_ref, b2_ref,
               out_ref, loss_ref):
    i = pl.program_id(0)

    @pl.when(i == 0)
    def _init():
        loss_ref[...] = jnp.zeros_like(loss_ref)

    hid = jnp.maximum(a_ref[...] + b_ref[...] + b1_ref[...], 0.0)
    rec = hid @ w2_ref[...] + b2_ref[...]
    out_ref[...] = rec
    d = rec - attr_ref[...]
    part = jnp.sum(d * d * w_ref[...], axis=0, keepdims=True)
    loss_ref[...] += part


def _head(gath, attr, b1, W2, b2):
    grid = (E // BE,)
    nblk = E // BE
    ein = lambda i: (i, 0)
    hd_map = lambda i: (i + nblk, 0)
    full = lambda i: (0, 0)
    return pl.pallas_call(
        _head_body,
        grid=grid,
        in_specs=[
            pl.BlockSpec((BE, H), ein),
            pl.BlockSpec((BE, H), hd_map),
            pl.BlockSpec((BE, EDGE_IN), ein),
            pl.BlockSpec((BE, 1), ein),
            pl.BlockSpec((1, H), full),
            pl.BlockSpec((H, EDGE_IN), full),
            pl.BlockSpec((1, EDGE_IN), full),
        ],
        out_specs=[
            pl.BlockSpec((BE, EDGE_IN), ein),
            pl.BlockSpec((1, EDGE_IN), full),
        ],
        out_shape=[
            jax.ShapeDtypeStruct((E, EDGE_IN), jnp.float32),
            jax.ShapeDtypeStruct((1, EDGE_IN), jnp.float32),
        ],
        compiler_params=pltpu.CompilerParams(
            dimension_semantics=("arbitrary",)),
    )(gath, gath, attr, jnp.asarray(_W_LOSS), b1, W2, b2)


def _proj_body(h_ref, w1_ref, out_ref):
    out_ref[...] = h_ref[...] @ w1_ref[0]


def _proj(node_emb, W1ab):
    nblk = N // BN
    grid = (2, nblk)
    return pl.pallas_call(
        _proj_body,
        grid=grid,
        in_specs=[
            pl.BlockSpec((BN, H), lambda g, i: (i, 0)),
            pl.BlockSpec((1, H, H), lambda g, i: (g, 0, 0)),
        ],
        out_specs=pl.BlockSpec((BN, H), lambda g, i: (g * nblk + i, 0)),
        out_shape=jax.ShapeDtypeStruct((2 * N, H), jnp.float32),
        compiler_params=pltpu.CompilerParams(
            dimension_semantics=("arbitrary", "arbitrary")),
    )(node_emb, W1ab)


def kernel(x, edge_index, edge_attr, W_node, b_node, W_edge, b_edge,
           W_msg, b_msg, W_upd, b_upd, W_h1, b_h1, W_h2, b_h2, mask_token):
    src = edge_index[0]
    dst = edge_index[1]

    tok = jnp.asarray(_IS_TOK)
    urand = jnp.asarray(_USE_RAND)
    masked_attr = (1.0 - tok - urand) * edge_attr \
        + tok * mask_token[None, :] + urand * jnp.asarray(_RAND_CONST)

    iota = lax.iota(jnp.int32, E)
    sdst, perm = lax.sort_key_val(dst, iota)
    ssrc = jnp.take(src, perm, axis=0, mode='clip')
    attr_s = jnp.take(masked_attr, perm, axis=0, mode='clip')

    idx_cat = jnp.concatenate([ssrc, sdst])
    idx_head = jnp.concatenate([src, dst + N])

    h = jax.nn.relu(x @ W_node + b_node)

    b_msg2 = b_msg.reshape(L, 1, H)
    b_upd2 = b_upd.reshape(L, 1, H)
    bedge2 = b_edge.reshape(1, H)

    for l in range(L):
        gath = _sc_gather(h, idx_cat, 2 * E)
        Ws = W_msg[l, :H, :]
        Wd = W_msg[l, H:2 * H, :]
        We = W_msg[l, 2 * H:, :]
        m = _msg_matmul(gath, attr_s, Ws, Wd, We,
                        b_msg2[l], W_edge, bedge2)
        agg_ref = jax.new_ref(jnp.zeros(((N + _NPAD) * H,), jnp.float32))
        brow, bid = _sc_segsum(m, sdst, agg_ref)
        bid2 = bid.reshape(32, 16)
        br3 = brow.reshape(32, 2, H)
        agg_p = agg_ref[...].reshape(N + _NPAD, H) \
            .at[bid2[:, 0]].add(br3[:, 0, :]) \
            .at[bid2[:, 1]].add(br3[:, 1, :])
        h = _update(h, agg_p, W_upd[l, :H, :], W_upd[l, H:, :], b_upd2[l])

    node_emb = h

    projcat = _proj(node_emb, jnp.stack([W_h1[:H, :], W_h1[H:, :]]))
    gath_head = _sc_gather(projcat, idx_head, 2 * E)
    reconstructed, loss_part = _head(gath_head, edge_attr,
                                     b_h1.reshape(1, H), W_h2,
                                     b_h2.reshape(1, EDGE_IN))
    loss = jnp.sum(loss_part)
    return (loss, reconstructed, node_emb)
